# Initial kernel scaffold; baseline (speedup 1.0000x reference)
import functools

import jax
import jax.numpy as jnp
from jax import lax
from jax.experimental import pallas as pl
from jax.experimental.pallas import tpu as pltpu
from jax.experimental.pallas import tpu_sc as plsc

VOCAB = 1000000
DIM = 32
B, T, L = 4096, 50, 20
N = B * T
NC, NS = 2, 16
NW = NC * NS
ROWS_PER_W = N // NW
R = 64
NCHUNK = ROWS_PER_W // R
IDX_PER_CHUNK = R * L
GATHER_W = 128
NGATHER = IDX_PER_CHUNK // GATHER_W


def _body(idx_hbm, table_hbm, out_hbm, idx_v, rows_v, out_v, sem):
    wid = lax.axis_index("s") * NC + lax.axis_index("c")

    def chunk(g, _):
        base = (wid * NCHUNK + g) * R
        pltpu.sync_copy(idx_hbm.at[pl.ds(base * L, IDX_PER_CHUNK)], idx_v)
        copies = []
        for k in range(NGATHER):
            copies.append(pltpu.async_copy(
                table_hbm.at[idx_v.at[pl.ds(k * GATHER_W, GATHER_W)]],
                rows_v.at[pl.ds(k * GATHER_W, GATHER_W)],
                sem,
            ))
        for c in copies:
            c.wait()

        def reduce_row(r, _):
            j = r * L
            a0 = rows_v[j, pl.ds(0, 16)]
            a1 = rows_v[j, pl.ds(16, 16)]
            for l in range(1, L):
                a0 += rows_v[j + l, pl.ds(0, 16)]
                a1 += rows_v[j + l, pl.ds(16, 16)]
            out_v[r, pl.ds(0, 16)] = a0
            out_v[r, pl.ds(16, 16)] = a1
            return 0

        lax.fori_loop(0, R, reduce_row, 0)
        pltpu.sync_copy(out_v, out_hbm.at[pl.ds(base, R), :])
        return 0

    lax.fori_loop(0, NCHUNK, chunk, 0)


@jax.jit
def _run(idx_flat, table):
    mesh = plsc.VectorSubcoreMesh(core_axis_name="c", subcore_axis_name="s")
    return pl.kernel(
        _body,
        out_type=jax.ShapeDtypeStruct((N, DIM), jnp.float32),
        mesh=mesh,
        scratch_types=[
            pltpu.VMEM((IDX_PER_CHUNK,), jnp.int32),
            pltpu.VMEM((IDX_PER_CHUNK, DIM), jnp.float32),
            pltpu.VMEM((R, DIM), jnp.float32),
            pltpu.SemaphoreType.DMA,
        ],
    )(idx_flat, table)


def kernel(sentences, table):
    idx_flat = sentences.reshape(-1)
    out = _run(idx_flat, table)
    return out.reshape(B, T, DIM)


# speedup vs baseline: 14.2176x; 14.2176x over previous
import functools

import jax
import jax.numpy as jnp
from jax import lax
from jax.experimental import pallas as pl
from jax.experimental.pallas import tpu as pltpu
from jax.experimental.pallas import tpu_sc as plsc

VOCAB = 1000000
DIM = 32
B, T, L = 4096, 50, 20
N = B * T
NC, NS = 2, 16
NW = NC * NS
ROWS_PER_W = N // NW
R = 64
NCHUNK = ROWS_PER_W // R
IDX_PER_CHUNK = R * L
GATHER_W = 128
NGATHER = IDX_PER_CHUNK // GATHER_W


def _body(idx_hbm, table_hbm, out_hbm, idx_v, rows_v, out_v, sem):
    wid = lax.axis_index("s") * NC + lax.axis_index("c")

    def chunk(g, _):
        base = (wid * NCHUNK + g) * R
        pltpu.sync_copy(idx_hbm.at[pl.ds(base * L, IDX_PER_CHUNK)], idx_v)
        copies = []
        for k in range(NGATHER):
            copies.append(pltpu.async_copy(
                table_hbm.at[idx_v.at[pl.ds(k * GATHER_W, GATHER_W)]],
                rows_v.at[pl.ds(k * GATHER_W, GATHER_W)],
                sem,
            ))
        for c in copies:
            c.wait()

        def reduce_row(r, _):
            j = r * L
            a0 = rows_v[j, pl.ds(0, 16)]
            a1 = rows_v[j, pl.ds(16, 16)]
            for l in range(1, L):
                a0 += rows_v[j + l, pl.ds(0, 16)]
                a1 += rows_v[j + l, pl.ds(16, 16)]
            out_v[r, pl.ds(0, 16)] = a0
            out_v[r, pl.ds(16, 16)] = a1
            return 0

        lax.fori_loop(0, R, reduce_row, 0)
        pltpu.sync_copy(out_v, out_hbm.at[pl.ds(base, R), :])
        return 0

    lax.fori_loop(0, NCHUNK, chunk, 0)


@jax.jit
def _run(idx_flat, table):
    mesh = plsc.VectorSubcoreMesh(core_axis_name="c", subcore_axis_name="s")
    return pl.kernel(
        _body,
        out_type=jax.ShapeDtypeStruct((N, DIM), jnp.float32),
        mesh=mesh,
        scratch_types=[
            pltpu.VMEM((IDX_PER_CHUNK,), jnp.int32),
            pltpu.VMEM((IDX_PER_CHUNK, DIM), jnp.float32),
            pltpu.VMEM((R, DIM), jnp.float32),
            pltpu.SemaphoreType.DMA,
        ],
        compiler_params=pltpu.CompilerParams(use_tc_tiling_on_sc=False),
    )(idx_flat, table)


def kernel(sentences, table):
    idx_flat = sentences.reshape(-1)
    out = _run(idx_flat, table)
    return out.reshape(B, T, DIM)


# speedup vs baseline: 17.7267x; 1.2468x over previous
import jax
import jax.numpy as jnp
from jax import lax
from jax.experimental import pallas as pl
from jax.experimental.pallas import tpu as pltpu
from jax.experimental.pallas import tpu_sc as plsc

VOCAB = 1000000
DIM = 32
B, T, L = 4096, 50, 20
N = B * T
NC, NS = 2, 16
NW = NC * NS
ROWS_PER_W = N // NW
R = 128
NCHUNK = ROWS_PER_W // R


def _body(idx_hbm, table_hbm, out_hbm, idx_v, acc_v, sem):
    wid = lax.axis_index("s") * NC + lax.axis_index("c")
    zeros = jnp.zeros((16,), jnp.float32)

    def chunk(g, _):
        base = (wid * NCHUNK + g) * R
        pltpu.sync_copy(idx_hbm.at[:, pl.ds(base, R)], idx_v)

        def zero_row(r, _):
            acc_v[r, pl.ds(0, 16)] = zeros
            acc_v[r, pl.ds(16, 16)] = zeros
            return 0

        lax.fori_loop(0, R, zero_row, 0)

        copies = []
        for l in range(L):
            copies.append(pltpu.async_copy(
                table_hbm.at[idx_v.at[l]], acc_v, sem, add=True))
        for c in copies:
            c.wait()
        pltpu.sync_copy(acc_v, out_hbm.at[pl.ds(base, R), :])
        return 0

    lax.fori_loop(0, NCHUNK, chunk, 0)


@jax.jit
def _run(idx_t, table):
    mesh = plsc.VectorSubcoreMesh(core_axis_name="c", subcore_axis_name="s")
    return pl.kernel(
        _body,
        out_type=jax.ShapeDtypeStruct((N, DIM), jnp.float32),
        mesh=mesh,
        scratch_types=[
            pltpu.VMEM((L, R), jnp.int32),
            pltpu.VMEM((R, DIM), jnp.float32),
            pltpu.SemaphoreType.DMA,
        ],
        compiler_params=pltpu.CompilerParams(use_tc_tiling_on_sc=False),
    )(idx_t, table)


def kernel(sentences, table):
    idx_t = sentences.reshape(N, L).T
    out = _run(idx_t, table)
    return out.reshape(B, T, DIM)


# speedup vs baseline: 17.9212x; 1.0110x over previous
import jax
import jax.numpy as jnp
from jax import lax
from jax.experimental import pallas as pl
from jax.experimental.pallas import tpu as pltpu
from jax.experimental.pallas import tpu_sc as plsc

VOCAB = 1000000
DIM = 32
B, T, L = 4096, 50, 20
N = B * T
NC, NS = 2, 16
NW = NC * NS
ROWS_PER_W = N // NW
R = 128
NCHUNK = ROWS_PER_W // R
IDXW = L * R


def _body(idx_hbm, table_hbm, out_hbm,
          idx0, idx1, acc0, acc1,
          gsem0, gsem1, isem0, isem1, osem0, osem1):
    wid = lax.axis_index("s") * NC + lax.axis_index("c")
    zeros = jnp.zeros((16,), jnp.float32)

    def zero(acc):
        def zrow(r, _):
            acc[r, pl.ds(0, 16)] = zeros
            acc[r, pl.ds(16, 16)] = zeros
            return 0
        lax.fori_loop(0, R, zrow, 0)

    def fire(idx, acc, gsem):
        return [pltpu.async_copy(table_hbm.at[idx.at[l]], acc, gsem, add=True)
                for l in range(L)]

    def drain(copies):
        for c in copies:
            c.wait()

    def idx_async(g, idx, isem):
        return pltpu.async_copy(idx_hbm.at[wid, g], idx, isem)

    def out_async(g, acc, osem):
        base = (wid * NCHUNK + g) * R
        return pltpu.async_copy(acc, out_hbm.at[pl.ds(base, R), :], osem)

    def wait_idx(idx, isem):
        pltpu.make_async_copy(idx_hbm.at[wid, 0], idx, isem).wait()

    def wait_out(acc, osem):
        pltpu.make_async_copy(acc, out_hbm.at[pl.ds(0, R), :], osem).wait()

    idx_async(0, idx0, isem0).wait()
    zero(acc0)
    c0 = fire(idx0, acc0, gsem0)
    idx_async(1, idx1, isem1)

    def step(it, carry):
        g = it * 2
        wait_idx(idx1, isem1)

        @pl.when(it > 0)
        def _():
            wait_out(acc1, osem1)
        zero(acc1)
        c1 = fire(idx1, acc1, gsem1)
        drain(c0)
        out_async(g, acc0, osem0)

        @pl.when(g + 2 < NCHUNK)
        def _():
            idx_async(g + 2, idx0, isem0)
            wait_idx(idx0, isem0)
            wait_out(acc0, osem0)
            zero(acc0)
            fire(idx0, acc0, gsem0)
        drain(c1)
        out_async(g + 1, acc1, osem1)

        @pl.when(g + 3 < NCHUNK)
        def _():
            idx_async(g + 3, idx1, isem1)
        return carry

    lax.fori_loop(0, NCHUNK // 2, step, 0)
    wait_out(acc0, osem0)
    wait_out(acc1, osem1)


@jax.jit
def _run(idx_blk, table):
    mesh = plsc.VectorSubcoreMesh(core_axis_name="c", subcore_axis_name="s")
    return pl.kernel(
        _body,
        out_type=jax.ShapeDtypeStruct((N, DIM), jnp.float32),
        mesh=mesh,
        scratch_types=[
            pltpu.VMEM((L, R), jnp.int32),
            pltpu.VMEM((L, R), jnp.int32),
            pltpu.VMEM((R, DIM), jnp.float32),
            pltpu.VMEM((R, DIM), jnp.float32),
            pltpu.SemaphoreType.DMA,
            pltpu.SemaphoreType.DMA,
            pltpu.SemaphoreType.DMA,
            pltpu.SemaphoreType.DMA,
            pltpu.SemaphoreType.DMA,
            pltpu.SemaphoreType.DMA,
        ],
        compiler_params=pltpu.CompilerParams(use_tc_tiling_on_sc=False),
    )(idx_blk, table)


def kernel(sentences, table):
    idx_blk = sentences.reshape(NW, NCHUNK, R, L).transpose(0, 1, 3, 2)
    out = _run(idx_blk, table)
    return out.reshape(B, T, DIM)


# speedup vs baseline: 18.3934x; 1.0264x over previous
import jax
import jax.numpy as jnp
from jax import lax
from jax.experimental import pallas as pl
from jax.experimental.pallas import tpu as pltpu
from jax.experimental.pallas import tpu_sc as plsc

VOCAB = 1000000
DIM = 32
B, T, L = 4096, 50, 20
N = B * T
NC, NS = 2, 16
NW = NC * NS
ROWS_PER_W = N // NW
R = 800
NCHUNK = ROWS_PER_W // R
IDXW = L * R


def _body(idx_hbm, table_hbm, out_hbm,
          idx0, idx1, acc0, acc1,
          gsem0, gsem1, isem0, isem1, osem0, osem1):
    wid = lax.axis_index("s") * NC + lax.axis_index("c")
    zeros = jnp.zeros((16,), jnp.float32)

    def zero(acc):
        def zrow(r, _):
            acc[r, pl.ds(0, 16)] = zeros
            acc[r, pl.ds(16, 16)] = zeros
            return 0
        lax.fori_loop(0, R, zrow, 0)

    def fire(idx, acc, gsem):
        return [pltpu.async_copy(table_hbm.at[idx.at[l]], acc, gsem, add=True)
                for l in range(L)]

    def drain(copies):
        for c in copies:
            c.wait()

    def idx_async(g, idx, isem):
        return pltpu.async_copy(idx_hbm.at[wid, g], idx, isem)

    def out_async(g, acc, osem):
        base = (wid * NCHUNK + g) * R
        return pltpu.async_copy(acc, out_hbm.at[pl.ds(base, R), :], osem)

    def wait_idx(idx, isem):
        pltpu.make_async_copy(idx_hbm.at[wid, 0], idx, isem).wait()

    def wait_out(acc, osem):
        pltpu.make_async_copy(acc, out_hbm.at[pl.ds(0, R), :], osem).wait()

    idx_async(0, idx0, isem0).wait()
    zero(acc0)
    c0 = fire(idx0, acc0, gsem0)
    idx_async(1, idx1, isem1)

    def step(it, carry):
        g = it * 2
        wait_idx(idx1, isem1)

        @pl.when(it > 0)
        def _():
            wait_out(acc1, osem1)
        zero(acc1)
        c1 = fire(idx1, acc1, gsem1)
        drain(c0)
        out_async(g, acc0, osem0)

        @pl.when(g + 2 < NCHUNK)
        def _():
            idx_async(g + 2, idx0, isem0)
            wait_idx(idx0, isem0)
            wait_out(acc0, osem0)
            zero(acc0)
            fire(idx0, acc0, gsem0)
        drain(c1)
        out_async(g + 1, acc1, osem1)

        @pl.when(g + 3 < NCHUNK)
        def _():
            idx_async(g + 3, idx1, isem1)
        return carry

    lax.fori_loop(0, NCHUNK // 2, step, 0)
    wait_out(acc0, osem0)
    wait_out(acc1, osem1)


@jax.jit
def _run(idx_blk, table):
    mesh = plsc.VectorSubcoreMesh(core_axis_name="c", subcore_axis_name="s")
    return pl.kernel(
        _body,
        out_type=jax.ShapeDtypeStruct((N, DIM), jnp.float32),
        mesh=mesh,
        scratch_types=[
            pltpu.VMEM((L, R), jnp.int32),
            pltpu.VMEM((L, R), jnp.int32),
            pltpu.VMEM((R, DIM), jnp.float32),
            pltpu.VMEM((R, DIM), jnp.float32),
            pltpu.SemaphoreType.DMA,
            pltpu.SemaphoreType.DMA,
            pltpu.SemaphoreType.DMA,
            pltpu.SemaphoreType.DMA,
            pltpu.SemaphoreType.DMA,
            pltpu.SemaphoreType.DMA,
        ],
        compiler_params=pltpu.CompilerParams(use_tc_tiling_on_sc=False),
    )(idx_blk, table)


def kernel(sentences, table):
    idx_blk = sentences.reshape(NW, NCHUNK, R, L).transpose(0, 1, 3, 2)
    out = _run(idx_blk, table)
    return out.reshape(B, T, DIM)
